# baseline (speedup 1.0000x reference)
import jax
import jax.numpy as jnp
from jax.experimental import pallas as pl

B, N, DIN, DOUT, T = 2, 2048, 768, 768, 8
BLK = 512


def _hetero_block_kernel(tv_ref, x_ref, W_ref, b_ref, out_ref):
    x = x_ref[0]
    tv = tv_ref[0]
    acc = jnp.zeros((BLK, DOUT), jnp.float32)
    for t in range(T):
        w = W_ref[t]
        y = jax.lax.dot_general(
            x, w, (((1,), (1,)), ((), ())),
            preferred_element_type=jnp.float32)
        y = y + b_ref[t][None, :]
        mask = (tv == t)
        acc = jnp.where(mask, y, acc)
    out_ref[0] = acc


def kernel(x, type_vec, W, b):
    nblk = N // BLK
    x = x.astype(jnp.bfloat16)
    W = W.astype(jnp.bfloat16)
    tv3 = type_vec.reshape(nblk, BLK, 1)
    grid = (B, nblk)
    out = pl.pallas_call(
        _hetero_block_kernel,
        grid=grid,
        in_specs=[
            pl.BlockSpec((1, BLK, 1), lambda bi, j: (j, 0, 0)),
            pl.BlockSpec((1, BLK, DIN), lambda bi, j: (bi, j, 0)),
            pl.BlockSpec((T, DOUT, DIN), lambda bi, j: (0, 0, 0)),
            pl.BlockSpec((T, DOUT), lambda bi, j: (0, 0)),
        ],
        out_specs=pl.BlockSpec((1, BLK, DOUT), lambda bi, j: (bi, j, 0)),
        out_shape=jax.ShapeDtypeStruct((B, N, DOUT), jnp.float32),
    )(tv3, x, W, b)
    return out


# speedup vs baseline: 1.3028x; 1.3028x over previous
import jax
import jax.numpy as jnp
from jax.experimental import pallas as pl

B, N, DIN, DOUT, T = 2, 2048, 768, 768, 8
BLK = 1024


def _hetero_block_kernel(tv_ref, x_ref, W_ref, b_ref, out_ref):
    x = x_ref[0].astype(jnp.bfloat16)
    tv = tv_ref[0]
    acc = jnp.zeros((BLK, DOUT), jnp.float32)
    for t in range(T):
        w = W_ref[t].astype(jnp.bfloat16)
        y = jax.lax.dot_general(
            x, w, (((1,), (1,)), ((), ())),
            preferred_element_type=jnp.float32)
        y = y + b_ref[t][None, :]
        mask = (tv == t)
        acc = jnp.where(mask, y, acc)
    out_ref[0] = acc


def kernel(x, type_vec, W, b):
    nblk = N // BLK
    tv3 = type_vec.reshape(nblk, BLK, 1)
    grid = (B, nblk)
    out = pl.pallas_call(
        _hetero_block_kernel,
        grid=grid,
        in_specs=[
            pl.BlockSpec((1, BLK, 1), lambda bi, j: (j, 0, 0)),
            pl.BlockSpec((1, BLK, DIN), lambda bi, j: (bi, j, 0)),
            pl.BlockSpec((T, DOUT, DIN), lambda bi, j: (0, 0, 0)),
            pl.BlockSpec((T, DOUT), lambda bi, j: (0, 0)),
        ],
        out_specs=pl.BlockSpec((1, BLK, DOUT), lambda bi, j: (bi, j, 0)),
        out_shape=jax.ShapeDtypeStruct((B, N, DOUT), jnp.float32),
    )(tv3, x, W, b)
    return out
